# baseline (speedup 1.0000x reference)
import functools

import jax
import jax.numpy as jnp
from jax import lax
from jax.experimental import pallas as pl
from jax.experimental.pallas import tpu as pltpu
from jax.experimental.pallas import tpu_sc as plsc

_LN_EPS = 1e-5
_CH = 128
_NC = 2
_NS = 16


def _ln(v, g, bt):
    mu = jnp.mean(v, axis=-1, keepdims=True)
    var = jnp.mean((v - mu) * (v - mu), axis=-1, keepdims=True)
    return (v - mu) * lax.rsqrt(var + _LN_EPS) * g + bt


def _y_body(xp_ref, xn_ref, g_ref, bt_ref, wn_ref, y_ref):
    g = g_ref[...]
    bt = bt_ref[...]
    x = jnp.concatenate([_ln(xp_ref[...], g, bt), _ln(xn_ref[...], g, bt)],
                        axis=1)
    y = jnp.dot(x, wn_ref[...], preferred_element_type=jnp.float32)
    y = y.astype(jnp.bfloat16)
    d_half = y.shape[1] // 2
    y_ref[0] = y[:, :d_half]
    y_ref[1] = y[:, d_half:]


def _root_body(xp_ref, xn_ref, g_ref, bt_ref, wr_ref, b_ref, root_ref):
    g = g_ref[...]
    bt = bt_ref[...]
    x = jnp.concatenate([_ln(xp_ref[...], g, bt), _ln(xn_ref[...], g, bt)],
                        axis=1)
    root_ref[...] = (
        jnp.dot(x, wr_ref[...], preferred_element_type=jnp.float32) + b_ref[...]
    )


def _combine_body(root_ref, p00_ref, p01_ref, p10_ref, p11_ref, out_ref):
    a0 = p00_ref[0, 0].astype(jnp.float32) + p01_ref[0, 0].astype(jnp.float32)
    a1 = p10_ref[0, 0].astype(jnp.float32) + p11_ref[0, 0].astype(jnp.float32)
    out_ref[...] = root_ref[...] + jnp.concatenate([a0, a1], axis=1)


def _make_sc_kernel(n_pad, k, dh):
    rows_per_sub = n_pad // _NS
    mesh = plsc.VectorSubcoreMesh(core_axis_name="c", subcore_axis_name="s")
    nbuf = 6

    @functools.partial(
        pl.kernel,
        out_type=jax.ShapeDtypeStruct((_NC, 2, n_pad, dh), jnp.bfloat16),
        mesh=mesh,
        scratch_types=[
            pltpu.VMEM((k, _CH), jnp.int32),
            pltpu.VMEM((k, _CH), jnp.int32),
            pltpu.VMEM((nbuf, _CH, dh), jnp.bfloat16),
            pltpu.VMEM_SHARED((2, n_pad, dh), jnp.bfloat16),
            pltpu.SemaphoreType.DMA,
            pltpu.SemaphoreType.DMA,
        ],
        compiler_params=pltpu.CompilerParams(use_tc_tiling_on_sc=False),
    )
    def sc_kernel(y_hbm, ei_hbm, zeros_hbm, out_hbm,
                  src_v, dst_v, rows_v, acc, gsem, ssem):
        c = lax.axis_index("c")
        s = lax.axis_index("s")
        pltpu.sync_copy(ei_hbm.at[0, s], src_v)
        pltpu.sync_copy(ei_hbm.at[1, s], dst_v)
        row0 = s * rows_per_sub
        for p2 in range(2):
            pltpu.sync_copy(zeros_hbm.at[pl.ds(row0, rows_per_sub)],
                            acc.at[p2, pl.ds(row0, rows_per_sub)])
        plsc.subcore_barrier()

        def gather(j, slot):
            pltpu.async_copy(y_hbm.at[c].at[src_v.at[j]], rows_v.at[slot],
                             gsem)

        def scatter(j, slot):
            pltpu.async_copy(rows_v.at[slot],
                             acc.at[lax.rem(j, 2)].at[dst_v.at[j]], ssem,
                             add=True)

        def wait(sem):
            pltpu.make_async_copy(zeros_hbm.at[pl.ds(0, _CH)],
                                  rows_v.at[0], sem).wait()

        for j in range(min(4, k)):
            gather(j, j % nbuf)

        def body(j, carry):
            @pl.when(j >= 2)
            def _():
                wait(ssem)
            @pl.when(j + 4 < k)
            def _():
                gather(j + 4, lax.rem(j + 4, nbuf))
            wait(gsem)
            scatter(j, lax.rem(j, nbuf))
            return carry

        lax.fori_loop(0, k, body, 0)
        for _ in range(min(2, k)):
            wait(ssem)
        plsc.subcore_barrier()
        for p2 in range(2):
            pltpu.sync_copy(acc.at[p2, pl.ds(row0, rows_per_sub)],
                            out_hbm.at[c, p2, pl.ds(row0, rows_per_sub)])

    return sc_kernel


def kernel(x_prev, x_same, x_next, edge_index, ln_gamma, ln_beta,
           W_root, W_neigh, b):
    n, d_prev = x_prev.shape
    d_out = W_root.shape[1]
    dh = d_out // 2
    e = edge_index.shape[1]

    k = -(-e // (_NS * _CH))
    e_pad = _NS * _CH * k
    ny = n + 16
    n_pad = -(-(n + 1) // (_NS * 8)) * (_NS * 8)

    g2 = ln_gamma.reshape(1, -1)
    bt2 = ln_beta.reshape(1, -1)
    bn = 2000
    grid = (n // bn,)
    row_spec = pl.BlockSpec((bn, d_prev), lambda i: (i, 0))
    vec_spec = pl.BlockSpec((1, d_prev), lambda i: (0, 0))

    y = pl.pallas_call(
        _y_body,
        grid=grid,
        in_specs=[row_spec, row_spec, vec_spec, vec_spec,
                  pl.BlockSpec(W_neigh.shape, lambda i: (0, 0))],
        out_specs=pl.BlockSpec((2, bn, dh), lambda i: (0, i, 0)),
        out_shape=jax.ShapeDtypeStruct((2, ny, dh), jnp.bfloat16),
    )(x_prev, x_next, g2, bt2, W_neigh)

    npad_e = e_pad - e
    pad_src = jnp.full((1, npad_e), n, jnp.int32)
    pad_dst = (n + jnp.arange(npad_e, dtype=jnp.int32) % (n_pad - n))[None]
    ei = jnp.concatenate([edge_index, jnp.concatenate([pad_src, pad_dst], 0)],
                         axis=1)
    ei = ei.reshape(2, _NS, k, _CH)
    zeros = jnp.zeros((n_pad, dh), jnp.bfloat16)

    partials = _make_sc_kernel(n_pad, k, dh)(y, ei, zeros)

    root = pl.pallas_call(
        _root_body,
        grid=grid,
        in_specs=[row_spec, row_spec, vec_spec, vec_spec,
                  pl.BlockSpec(W_root.shape, lambda i: (0, 0)),
                  pl.BlockSpec((1, d_out), lambda i: (0, 0))],
        out_specs=pl.BlockSpec((bn, d_out), lambda i: (i, 0)),
        out_shape=jax.ShapeDtypeStruct((n, d_out), jnp.float32),
    )(x_prev, x_next, g2, bt2, W_root, b.reshape(1, -1))

    out = pl.pallas_call(
        _combine_body,
        grid=grid,
        in_specs=[
            pl.BlockSpec((bn, d_out), lambda i: (i, 0)),
            pl.BlockSpec((1, 1, bn, dh), lambda i: (0, 0, i, 0)),
            pl.BlockSpec((1, 1, bn, dh), lambda i: (0, 1, i, 0)),
            pl.BlockSpec((1, 1, bn, dh), lambda i: (1, 0, i, 0)),
            pl.BlockSpec((1, 1, bn, dh), lambda i: (1, 1, i, 0)),
        ],
        out_specs=pl.BlockSpec((bn, d_out), lambda i: (i, 0)),
        out_shape=jax.ShapeDtypeStruct((n, d_out), jnp.float32),
    )(root, partials, partials, partials, partials)
    return out


# speedup vs baseline: 1.1593x; 1.1593x over previous
import functools

import jax
import jax.numpy as jnp
from jax import lax
from jax.experimental import pallas as pl
from jax.experimental.pallas import tpu as pltpu
from jax.experimental.pallas import tpu_sc as plsc

_LN_EPS = 1e-5
_CH = 128
_NC = 2
_NS = 16


def _ln(v, g, bt):
    mu = jnp.mean(v, axis=-1, keepdims=True)
    var = jnp.mean((v - mu) * (v - mu), axis=-1, keepdims=True)
    return (v - mu) * lax.rsqrt(var + _LN_EPS) * g + bt


def _y_body(xp_ref, xn_ref, g_ref, bt_ref, wn_ref, y_ref):
    g = g_ref[...]
    bt = bt_ref[...]
    x = jnp.concatenate([_ln(xp_ref[...], g, bt), _ln(xn_ref[...], g, bt)],
                        axis=1)
    y = jnp.dot(x, wn_ref[...], preferred_element_type=jnp.float32)
    y = y.astype(jnp.bfloat16)
    d_half = y.shape[1] // 2
    y_ref[0] = y[:, :d_half]
    y_ref[1] = y[:, d_half:]


def _root_body(xp_ref, xn_ref, g_ref, bt_ref, wr_ref, b_ref, root_ref):
    g = g_ref[...]
    bt = bt_ref[...]
    x = jnp.concatenate([_ln(xp_ref[...], g, bt), _ln(xn_ref[...], g, bt)],
                        axis=1)
    root_ref[...] = (
        jnp.dot(x, wr_ref[...], preferred_element_type=jnp.float32) + b_ref[...]
    )


def _combine_body(root_ref, p0_ref, p1_ref, out_ref):
    agg = jnp.concatenate([p0_ref[0], p1_ref[0]], axis=1).astype(jnp.float32)
    out_ref[...] = root_ref[...] + agg


def _make_sc_kernel(n_pad, k, dh):
    rows_per_sub = n_pad // _NS
    mesh = plsc.VectorSubcoreMesh(core_axis_name="c", subcore_axis_name="s")
    nbuf = 6

    @functools.partial(
        pl.kernel,
        out_type=jax.ShapeDtypeStruct((_NC, n_pad, dh), jnp.bfloat16),
        mesh=mesh,
        scratch_types=[
            pltpu.VMEM((k, _CH), jnp.int32),
            pltpu.VMEM((k, _CH), jnp.int32),
            pltpu.VMEM((nbuf, _CH, dh), jnp.bfloat16),
            pltpu.VMEM_SHARED((n_pad, dh), jnp.bfloat16),
            pltpu.SemaphoreType.DMA,
            pltpu.SemaphoreType.DMA,
        ],
        compiler_params=pltpu.CompilerParams(use_tc_tiling_on_sc=False),
    )
    def sc_kernel(y_hbm, ei_hbm, zeros_hbm, out_hbm,
                  src_v, dst_v, rows_v, acc, gsem, ssem):
        c = lax.axis_index("c")
        s = lax.axis_index("s")
        pltpu.sync_copy(ei_hbm.at[0, s], src_v)
        pltpu.sync_copy(ei_hbm.at[1, s], dst_v)
        row0 = s * rows_per_sub
        pltpu.sync_copy(zeros_hbm.at[pl.ds(row0, rows_per_sub)],
                        acc.at[pl.ds(row0, rows_per_sub)])
        plsc.subcore_barrier()

        def gather(j, slot):
            pltpu.async_copy(y_hbm.at[c].at[src_v.at[j]], rows_v.at[slot],
                             gsem)

        def scatter(j, slot):
            pltpu.async_copy(rows_v.at[slot], acc.at[dst_v.at[j]], ssem,
                             add=True)

        def wait(sem):
            pltpu.make_async_copy(zeros_hbm.at[pl.ds(0, _CH)],
                                  rows_v.at[0], sem).wait()

        for j in range(min(4, k)):
            gather(j, j % nbuf)

        def body(j, carry):
            @pl.when(j >= 2)
            def _():
                wait(ssem)
            @pl.when(j + 4 < k)
            def _():
                gather(j + 4, lax.rem(j + 4, nbuf))
            wait(gsem)
            scatter(j, lax.rem(j, nbuf))
            return carry

        lax.fori_loop(0, k, body, 0)
        for _ in range(min(2, k)):
            wait(ssem)
        plsc.subcore_barrier()
        pltpu.sync_copy(acc.at[pl.ds(row0, rows_per_sub)],
                        out_hbm.at[c, pl.ds(row0, rows_per_sub)])

    return sc_kernel


def kernel(x_prev, x_same, x_next, edge_index, ln_gamma, ln_beta,
           W_root, W_neigh, b):
    n, d_prev = x_prev.shape
    d_out = W_root.shape[1]
    dh = d_out // 2
    e = edge_index.shape[1]

    k = -(-e // (_NS * _CH))
    e_pad = _NS * _CH * k
    ny = n + 16
    n_pad = -(-(n + 1) // (_NS * 8)) * (_NS * 8)

    g2 = ln_gamma.reshape(1, -1)
    bt2 = ln_beta.reshape(1, -1)
    bn = 2000
    grid = (n // bn,)
    row_spec = pl.BlockSpec((bn, d_prev), lambda i: (i, 0))
    vec_spec = pl.BlockSpec((1, d_prev), lambda i: (0, 0))

    y = pl.pallas_call(
        _y_body,
        grid=grid,
        in_specs=[row_spec, row_spec, vec_spec, vec_spec,
                  pl.BlockSpec(W_neigh.shape, lambda i: (0, 0))],
        out_specs=pl.BlockSpec((2, bn, dh), lambda i: (0, i, 0)),
        out_shape=jax.ShapeDtypeStruct((2, ny, dh), jnp.bfloat16),
    )(x_prev, x_next, g2, bt2, W_neigh)

    npad_e = e_pad - e
    pad_src = jnp.full((1, npad_e), n, jnp.int32)
    pad_dst = (n + jnp.arange(npad_e, dtype=jnp.int32) % (n_pad - n))[None]
    ei = jnp.concatenate([edge_index, jnp.concatenate([pad_src, pad_dst], 0)],
                         axis=1)
    ei = ei.reshape(2, _NS, k, _CH)
    zeros = jnp.zeros((n_pad, dh), jnp.bfloat16)

    partials = _make_sc_kernel(n_pad, k, dh)(y, ei, zeros)

    root = pl.pallas_call(
        _root_body,
        grid=grid,
        in_specs=[row_spec, row_spec, vec_spec, vec_spec,
                  pl.BlockSpec(W_root.shape, lambda i: (0, 0)),
                  pl.BlockSpec((1, d_out), lambda i: (0, 0))],
        out_specs=pl.BlockSpec((bn, d_out), lambda i: (i, 0)),
        out_shape=jax.ShapeDtypeStruct((n, d_out), jnp.float32),
    )(x_prev, x_next, g2, bt2, W_root, b.reshape(1, -1))

    out = pl.pallas_call(
        _combine_body,
        grid=grid,
        in_specs=[
            pl.BlockSpec((bn, d_out), lambda i: (i, 0)),
            pl.BlockSpec((1, bn, dh), lambda i: (0, i, 0)),
            pl.BlockSpec((1, bn, dh), lambda i: (1, i, 0)),
        ],
        out_specs=pl.BlockSpec((bn, d_out), lambda i: (i, 0)),
        out_shape=jax.ShapeDtypeStruct((n, d_out), jnp.float32),
    )(root, partials, partials)
    return out


# speedup vs baseline: 1.2955x; 1.1175x over previous
import functools

import jax
import jax.numpy as jnp
from jax import lax
from jax.experimental import pallas as pl
from jax.experimental.pallas import tpu as pltpu
from jax.experimental.pallas import tpu_sc as plsc

_LN_EPS = 1e-5
_CH = 128
_NC = 2
_NS = 16


def _ln(v, g, bt):
    mu = jnp.mean(v, axis=-1, keepdims=True)
    var = jnp.mean((v - mu) * (v - mu), axis=-1, keepdims=True)
    return (v - mu) * lax.rsqrt(var + _LN_EPS) * g + bt


def _y_body(xp_ref, xn_ref, g_ref, bt_ref, wn_ref, y_ref):
    g = g_ref[...]
    bt = bt_ref[...]
    x = jnp.concatenate([_ln(xp_ref[...], g, bt), _ln(xn_ref[...], g, bt)],
                        axis=1)
    y = jnp.dot(x, wn_ref[...], preferred_element_type=jnp.float32)
    y = y.astype(jnp.bfloat16)
    d_half = y.shape[1] // 2
    y_ref[0] = y[:, :d_half]
    y_ref[1] = y[:, d_half:]


def _root_body(xp_ref, xn_ref, g_ref, bt_ref, wr_ref, b_ref, root_ref):
    g = g_ref[...]
    bt = bt_ref[...]
    x = jnp.concatenate([_ln(xp_ref[...], g, bt), _ln(xn_ref[...], g, bt)],
                        axis=1)
    root_ref[...] = (
        jnp.dot(x, wr_ref[...], preferred_element_type=jnp.float32) + b_ref[...]
    )


def _combine_body(root_ref, p0_ref, p1_ref, out_ref):
    agg = jnp.concatenate([p0_ref[0], p1_ref[0]], axis=1).astype(jnp.float32)
    out_ref[...] = root_ref[...] + agg


def _make_sc_kernel(n_pad, k, ch, dh):
    rows_per_sub = n_pad // _NS
    mesh = plsc.VectorSubcoreMesh(core_axis_name="c", subcore_axis_name="s")
    nbuf = 6

    @functools.partial(
        pl.kernel,
        out_type=jax.ShapeDtypeStruct((_NC, n_pad, dh), jnp.bfloat16),
        mesh=mesh,
        scratch_types=[
            pltpu.VMEM((k, ch), jnp.int32),
            pltpu.VMEM((k, ch), jnp.int32),
            pltpu.VMEM((nbuf, ch, dh), jnp.bfloat16),
            pltpu.VMEM_SHARED((n_pad, dh), jnp.bfloat16),
            pltpu.SemaphoreType.DMA,
            pltpu.SemaphoreType.DMA,
        ],
        compiler_params=pltpu.CompilerParams(use_tc_tiling_on_sc=False),
    )
    def sc_kernel(y_hbm, ei_hbm, zeros_hbm, out_hbm,
                  src_v, dst_v, rows_v, acc, gsem, ssem):
        c = lax.axis_index("c")
        s = lax.axis_index("s")
        pltpu.sync_copy(ei_hbm.at[0, s], src_v)
        pltpu.sync_copy(ei_hbm.at[1, s], dst_v)
        row0 = s * rows_per_sub
        pltpu.sync_copy(zeros_hbm.at[pl.ds(row0, rows_per_sub)],
                        acc.at[pl.ds(row0, rows_per_sub)])
        plsc.subcore_barrier()

        def gather(j, slot):
            pltpu.async_copy(y_hbm.at[c].at[src_v.at[j]], rows_v.at[slot],
                             gsem)

        def scatter(j, slot):
            pltpu.async_copy(rows_v.at[slot], acc.at[dst_v.at[j]], ssem,
                             add=True)

        def wait(sem):
            pltpu.make_async_copy(zeros_hbm.at[pl.ds(0, ch)],
                                  rows_v.at[0], sem).wait()

        for j in range(min(4, k)):
            gather(j, j % nbuf)

        def body(j, carry):
            @pl.when(j >= 2)
            def _():
                wait(ssem)
            @pl.when(j + 4 < k)
            def _():
                gather(j + 4, lax.rem(j + 4, nbuf))
            wait(gsem)
            scatter(j, lax.rem(j, nbuf))
            return carry

        lax.fori_loop(0, k, body, 0)
        for _ in range(min(2, k)):
            wait(ssem)
        plsc.subcore_barrier()
        pltpu.sync_copy(acc.at[pl.ds(row0, rows_per_sub)],
                        out_hbm.at[c, pl.ds(row0, rows_per_sub)])

    return sc_kernel


def kernel(x_prev, x_same, x_next, edge_index, ln_gamma, ln_beta,
           W_root, W_neigh, b):
    n, d_prev = x_prev.shape
    d_out = W_root.shape[1]
    dh = d_out // 2
    e = edge_index.shape[1]

    ch = None
    if e % _NS == 0:
        per_sub = e // _NS
        for cand in range(128, 7, -8):
            if per_sub % cand == 0:
                ch = cand
                break
    if ch is None:
        ch = _CH
    k = -(-e // (_NS * ch))
    e_pad = _NS * ch * k
    ny = n + 16
    n_pad = -(-(n + 1) // (_NS * 8)) * (_NS * 8)

    g2 = ln_gamma.reshape(1, -1)
    bt2 = ln_beta.reshape(1, -1)
    bn = 2000
    grid = (n // bn,)
    row_spec = pl.BlockSpec((bn, d_prev), lambda i: (i, 0))
    vec_spec = pl.BlockSpec((1, d_prev), lambda i: (0, 0))

    y = pl.pallas_call(
        _y_body,
        grid=grid,
        in_specs=[row_spec, row_spec, vec_spec, vec_spec,
                  pl.BlockSpec(W_neigh.shape, lambda i: (0, 0))],
        out_specs=pl.BlockSpec((2, bn, dh), lambda i: (0, i, 0)),
        out_shape=jax.ShapeDtypeStruct((2, ny, dh), jnp.bfloat16),
    )(x_prev, x_next, g2, bt2, W_neigh)

    npad_e = e_pad - e
    if npad_e:
        pad_src = jnp.full((1, npad_e), n, jnp.int32)
        pad_dst = (n + jnp.arange(npad_e, dtype=jnp.int32) % (n_pad - n))[None]
        ei = jnp.concatenate(
            [edge_index, jnp.concatenate([pad_src, pad_dst], 0)], axis=1)
    else:
        ei = edge_index
    ei = ei.reshape(2, _NS, k, ch)
    zeros = jnp.zeros((n_pad, dh), jnp.bfloat16)

    partials = _make_sc_kernel(n_pad, k, ch, dh)(y, ei, zeros)

    root = pl.pallas_call(
        _root_body,
        grid=grid,
        in_specs=[row_spec, row_spec, vec_spec, vec_spec,
                  pl.BlockSpec(W_root.shape, lambda i: (0, 0)),
                  pl.BlockSpec((1, d_out), lambda i: (0, 0))],
        out_specs=pl.BlockSpec((bn, d_out), lambda i: (i, 0)),
        out_shape=jax.ShapeDtypeStruct((n, d_out), jnp.float32),
    )(x_prev, x_next, g2, bt2, W_root, b.reshape(1, -1))

    out = pl.pallas_call(
        _combine_body,
        grid=grid,
        in_specs=[
            pl.BlockSpec((bn, d_out), lambda i: (i, 0)),
            pl.BlockSpec((1, bn, dh), lambda i: (0, i, 0)),
            pl.BlockSpec((1, bn, dh), lambda i: (1, i, 0)),
        ],
        out_specs=pl.BlockSpec((bn, d_out), lambda i: (i, 0)),
        out_shape=jax.ShapeDtypeStruct((n, d_out), jnp.float32),
    )(root, partials, partials)
    return out
